# baseline (speedup 1.0000x reference)
import jax
import jax.numpy as jnp
from jax.experimental import pallas as pl
from jax.experimental.pallas import tpu as pltpu

N = 16384
NUM_CLASSES = 1000
ROWS_PER_VROW = 16
WIDTH = ROWS_PER_VROW * NUM_CLASSES
NVROWS = N // ROWS_PER_VROW
BLOCK_VROWS = 64
GRID = NVROWS // BLOCK_VROWS
NBUF = 4
LANES = 128
NCOLS = WIDTH // LANES


def _onehot_block(x_ref, out_ref, scratch_ref, cg_ref, rk_ref, sems):
    i = pl.program_id(0)
    slot = jax.lax.rem(i, NBUF)

    @pl.when(i == 0)
    def _init_grids():
        l = jax.lax.broadcasted_iota(jnp.int32, (1, WIDTH), 1)
        rk = jnp.zeros((1, WIDTH), jnp.int32)
        for k in range(1, ROWS_PER_VROW):
            rk = rk + (l >= k * NUM_CLASSES).astype(jnp.int32)
        rk_ref[...] = rk
        cg_ref[...] = l - rk * NUM_CLASSES

    @pl.when(i >= NBUF)
    def _wait_slot():
        pltpu.make_async_copy(
            scratch_ref.at[slot],
            out_ref.at[pl.ds((i - NBUF) * BLOCK_VROWS, BLOCK_VROWS), :],
            sems.at[slot],
        ).wait()

    xb = x_ref[...]
    for v in range(NCOLS):
        lo, hi = v * LANES, (v + 1) * LANES
        k0 = lo // NUM_CLASSES
        k1 = (hi - 1) // NUM_CLASSES
        cg = cg_ref[0:1, lo:hi]
        x0 = xb[:, k0][:, None]
        if k0 == k1:
            av = (cg == x0).astype(jnp.int32)
        else:
            x1 = xb[:, k1][:, None]
            rk = rk_ref[0:1, lo:hi]
            xsel = jnp.where(rk == k0, x0, x1)
            av = (cg == xsel).astype(jnp.int32)
        scratch_ref[slot, :, lo:hi] = av

    for j in range(NBUF):
        @pl.when(slot == j)
        def _start(j=j):
            pltpu.make_async_copy(
                scratch_ref.at[j],
                out_ref.at[pl.ds(i * BLOCK_VROWS, BLOCK_VROWS), :],
                sems.at[j],
            ).start(priority=j % 2)

    @pl.when(i == GRID - 1)
    def _drain():
        for j in range(NBUF):
            step = GRID - NBUF + j
            s = step % NBUF
            pltpu.make_async_copy(
                scratch_ref.at[s],
                out_ref.at[pl.ds(step * BLOCK_VROWS, BLOCK_VROWS), :],
                sems.at[s],
            ).wait()


def kernel(x):
    x2 = x.reshape(NVROWS, ROWS_PER_VROW)
    y = pl.pallas_call(
        _onehot_block,
        grid=(GRID,),
        in_specs=[pl.BlockSpec((BLOCK_VROWS, ROWS_PER_VROW), lambda i: (i, 0))],
        out_specs=pl.BlockSpec(memory_space=pl.ANY),
        out_shape=jax.ShapeDtypeStruct((NVROWS, WIDTH), jnp.int32),
        scratch_shapes=[
            pltpu.VMEM((NBUF, BLOCK_VROWS, WIDTH), jnp.int32),
            pltpu.VMEM((1, WIDTH), jnp.int32),
            pltpu.VMEM((1, WIDTH), jnp.int32),
            pltpu.SemaphoreType.DMA((NBUF,)),
        ],
    )(x2)
    return y.reshape(N, NUM_CLASSES)
